# baseline (speedup 1.0000x reference)
import functools

import jax
import jax.numpy as jnp
from jax import lax
from jax.experimental import pallas as pl
from jax.experimental.pallas import tpu as pltpu
from jax.experimental.pallas import tpu_sc as plsc

D_MODEL = 1024
NUM_WORKERS = 32
CHUNK = 8
NBUF = 8
LEAD = 5


def _make_gather(batch):
    rows_per_worker = batch // NUM_WORKERS
    num_chunks = rows_per_worker // CHUNK
    num_groups = num_chunks // NBUF
    mesh = plsc.VectorSubcoreMesh(core_axis_name="c", subcore_axis_name="s")

    @functools.partial(
        pl.kernel,
        mesh=mesh,
        out_type=jax.ShapeDtypeStruct((batch, D_MODEL), jnp.float32),
        scratch_types=[
            pltpu.VMEM((rows_per_worker,), jnp.int32),
        ]
        + [pltpu.VMEM((CHUNK, D_MODEL), jnp.float32) for _ in range(NBUF)]
        + [pltpu.SemaphoreType.DMA for _ in range(2 * NBUF)],
    )
    def gather_kernel(table_hbm, idx_hbm, out_hbm, idx_v, *rest):
        bufs = rest[:NBUF]
        gsems = rest[NBUF:2 * NBUF]
        ssems = rest[2 * NBUF:]
        lead = LEAD
        wid = lax.axis_index("s") * 2 + lax.axis_index("c")
        base = wid * rows_per_worker
        pltpu.sync_copy(idx_hbm.at[pl.ds(base, rows_per_worker)], idx_v)

        for b in range(lead):
            pltpu.async_copy(
                table_hbm.at[idx_v.at[pl.ds(b * CHUNK, CHUNK)]], bufs[b], gsems[b]
            )

        def group(g, carry):
            goff = g * (NBUF * CHUNK)
            for b in range(NBUF):
                i_off = goff + b * CHUNK
                pltpu.make_async_copy(
                    table_hbm.at[idx_v.at[pl.ds(i_off, CHUNK)]], bufs[b], gsems[b]
                ).wait()
                pltpu.async_copy(
                    bufs[b], out_hbm.at[pl.ds(base + i_off, CHUNK)], ssems[b]
                )

                prev_off = i_off + (lead - NBUF) * CHUNK

                @pl.when(prev_off >= 0)
                def _():
                    pltpu.make_async_copy(
                        bufs[(b + lead) % NBUF],
                        out_hbm.at[pl.ds(base + prev_off, CHUNK)],
                        ssems[(b + lead) % NBUF],
                    ).wait()

                nxt_off = i_off + lead * CHUNK

                @pl.when(nxt_off < rows_per_worker)
                def _():
                    pltpu.async_copy(
                        table_hbm.at[idx_v.at[pl.ds(nxt_off, CHUNK)]],
                        bufs[(b + lead) % NBUF],
                        gsems[(b + lead) % NBUF],
                    )

            return carry

        lax.fori_loop(0, num_groups, group, 0)

        tail = NBUF - lead
        last = rows_per_worker - tail * CHUNK
        for b in range(tail):
            off = last + b * CHUNK
            pltpu.make_async_copy(
                bufs[(num_chunks - tail + b) % NBUF],
                out_hbm.at[pl.ds(base + off, CHUNK)],
                ssems[(num_chunks - tail + b) % NBUF],
            ).wait()

    return gather_kernel


def kernel(positions, pe):
    b, s = positions.shape
    n = b * s
    flat = positions.reshape(n)
    out = _make_gather(n)(pe, flat)
    return out.reshape(b, s, pe.shape[1])
